# baseline (speedup 1.0000x reference)
import functools

import jax
import jax.numpy as jnp
import numpy as np
from jax import lax
from jax.experimental import pallas as pl
from jax.experimental.pallas import tpu as pltpu
from jax.experimental.pallas import tpu_sc as plsc

_N = 10000
_D = 128
_HID = 256
_E = 320000

_NC = 2
_NS = 16
_NW = _NC * _NS
_NPAD = 10240
_RPT = _NPAD // _NS
_K = 64
_CPW = 160
_EPW = _K * _CPW
_EPAD = _NW * _EPW
_NB = 4

_BR = 1024
_GB = _NPAD // _BR
_BRO = 400
_GBO = _N // _BRO


def _sc_mesh():
    return plsc.VectorSubcoreMesh(core_axis_name="c", subcore_axis_name="s")


_CPH = _CPW // 2
_CPQ = _CPW // 4


@functools.partial(
    pl.kernel,
    mesh=_sc_mesh(),
    out_type=jax.ShapeDtypeStruct((_NC, _NPAD, _D), jnp.float32),
    scratch_types=[
        pltpu.VMEM((_CPH, _K), jnp.int32),
        pltpu.VMEM((_K, _D), jnp.float32),
        pltpu.VMEM_SHARED((_NPAD, _D), jnp.float32),
        pltpu.SemaphoreType.DMA((4,)),
    ],
)
def _degree_count(dst_hbm, zeros_hbm, out_hbm, dst_v, ones_v, degb, ssems):
    c = lax.axis_index("c")
    s = lax.axis_index("s")
    wid = s * _NC + c
    pltpu.sync_copy(zeros_hbm, degb.at[pl.ds(s * _RPT, _RPT)])
    ones = jnp.ones((16,), jnp.float32)

    def fill(i, carry):
        for w in range(_D // 16):
            ones_v[i, pl.ds(w * 16, 16)] = ones
        return carry

    lax.fori_loop(0, _K, fill, 0)
    plsc.subcore_barrier()

    for half in range(2):
        base = wid * _CPW + half * _CPH
        pltpu.sync_copy(dst_hbm.at[pl.ds(base, _CPH)], dst_v)

        def body(i, carry):
            for k in range(4):
                j = 4 * i + k

                @pl.when(j >= 4)
                def _():
                    pltpu.make_async_copy(
                        ones_v, degb.at[dst_v.at[j - 4]], ssems.at[k]).wait()

                pltpu.async_copy(ones_v, degb.at[dst_v.at[j]], ssems.at[k],
                                 add=True)
            return carry

        lax.fori_loop(0, _CPH // 4, body, 0)
        for k in range(4):
            j = _CPH - 4 + k
            pltpu.make_async_copy(
                ones_v, degb.at[dst_v.at[j]], ssems.at[k]).wait()
    plsc.subcore_barrier()
    pltpu.sync_copy(degb.at[pl.ds(s * _RPT, _RPT)],
                    out_hbm.at[c].at[pl.ds(s * _RPT, _RPT)])


@functools.partial(
    pl.kernel,
    mesh=_sc_mesh(),
    out_type=jax.ShapeDtypeStruct((_NC, _NPAD, _D), jnp.float32),
    scratch_types=[
        pltpu.VMEM((_CPQ, _K), jnp.int32),
        pltpu.VMEM((_CPQ, _K), jnp.int32),
        pltpu.VMEM((_NB, _K, _D), jnp.float32),
        pltpu.VMEM_SHARED((_NPAD, _D), jnp.float32),
        pltpu.SemaphoreType.DMA((_NB,)),
        pltpu.SemaphoreType.DMA((_NB,)),
    ],
)
def _edge_aggregate(g_hbm, src_hbm, dst_hbm, zeros_hbm, out_hbm,
                    src_v, dst_v, rows, agg, gsems, ssems):
    c = lax.axis_index("c")
    s = lax.axis_index("s")
    wid = s * _NC + c
    pltpu.sync_copy(zeros_hbm, agg.at[pl.ds(s * _RPT, _RPT)])
    plsc.subcore_barrier()

    def gather_start(j, k):
        pltpu.async_copy(g_hbm.at[src_v.at[j]], rows.at[k], gsems.at[k])

    def gather_wait(j, k):
        pltpu.make_async_copy(
            g_hbm.at[src_v.at[j]], rows.at[k], gsems.at[k]).wait()

    def scatter_start(j, k):
        pltpu.async_copy(rows.at[k], agg.at[dst_v.at[j]], ssems.at[k],
                         add=True)

    def scatter_wait(j, k):
        pltpu.make_async_copy(
            rows.at[k], agg.at[dst_v.at[j]], ssems.at[k]).wait()

    for quarter in range(4):
        base = wid * _CPW + quarter * _CPQ
        pltpu.sync_copy(src_hbm.at[pl.ds(base, _CPQ)], src_v)
        pltpu.sync_copy(dst_hbm.at[pl.ds(base, _CPQ)], dst_v)
        gather_start(0, 0)
        gather_start(1, 1)

        def body(i, carry):
            for k in range(_NB):
                j = _NB * i + k
                kn = (k + 2) % _NB

                @pl.when(j >= 2)
                def _():
                    scatter_wait(j - 2, kn)

                @pl.when(j + 2 < _CPQ)
                def _():
                    gather_start(j + 2, kn)

                gather_wait(j, k)
                scatter_start(j, k)
            return carry

        lax.fori_loop(0, _CPQ // _NB, body, 0)
        for k in range(2, _NB):
            scatter_wait(_CPQ - _NB + k, k)
    plsc.subcore_barrier()
    pltpu.sync_copy(agg.at[pl.ds(s * _RPT, _RPT)],
                    out_hbm.at[c].at[pl.ds(s * _RPT, _RPT)])


def _prep_body(degp_ref, x_ref, dinv_ref, g0_ref):
    d = degp_ref[...]
    deg = d[0, :, :1] + d[1, :, :1] + 1.0
    dinv = lax.rsqrt(deg)
    dinv_ref[...] = dinv
    g0_ref[...] = x_ref[...] * dinv


def _tc_prep(degp, x_p):
    return pl.pallas_call(
        _prep_body,
        grid=(_GB,),
        in_specs=[
            pl.BlockSpec((_NC, _BR, _D), lambda i: (0, i, 0)),
            pl.BlockSpec((_BR, _D), lambda i: (i, 0)),
        ],
        out_specs=[
            pl.BlockSpec((_BR, 1), lambda i: (i, 0)),
            pl.BlockSpec((_BR, _D), lambda i: (i, 0)),
        ],
        out_shape=[
            jax.ShapeDtypeStruct((_NPAD, 1), jnp.float32),
            jax.ShapeDtypeStruct((_NPAD, _D), jnp.float32),
        ],
    )(degp, x_p)


def _mid_body(r_ref, g0_ref, dinv_ref, w1_ref, b1_ref, w2_ref, g1_ref):
    r = r_ref[...]
    di = dinv_ref[...]
    z = (r[0] + r[1] + g0_ref[...]) * di
    h = jnp.dot(z, w1_ref[...], preferred_element_type=jnp.float32)
    h = jnp.maximum(h + b1_ref[...], 0.0)
    t = jnp.dot(h, w2_ref[...], preferred_element_type=jnp.float32)
    g1_ref[...] = t * di


def _tc_mid(r0, g0, dinv, W1, b1, W2):
    return pl.pallas_call(
        _mid_body,
        grid=(_GB,),
        in_specs=[
            pl.BlockSpec((_NC, _BR, _D), lambda i: (0, i, 0)),
            pl.BlockSpec((_BR, _D), lambda i: (i, 0)),
            pl.BlockSpec((_BR, 1), lambda i: (i, 0)),
            pl.BlockSpec((_D, _HID), lambda i: (0, 0)),
            pl.BlockSpec((1, _HID), lambda i: (0, 0)),
            pl.BlockSpec((_HID, _D), lambda i: (0, 0)),
        ],
        out_specs=pl.BlockSpec((_BR, _D), lambda i: (i, 0)),
        out_shape=jax.ShapeDtypeStruct((_NPAD, _D), jnp.float32),
    )(r0, g0, dinv, W1, b1, W2)


def _out_body(r_ref, g1_ref, dinv_ref, b2_ref, wl_ref, bl_ref, out_ref):
    r = r_ref[...]
    di = dinv_ref[...]
    z = (r[0] + r[1] + g1_ref[...]) * di + b2_ref[...]
    h = jnp.maximum(z, 0.0)
    o = jnp.dot(h, wl_ref[...], preferred_element_type=jnp.float32)
    out_ref[...] = o + bl_ref[...]


def _tc_out(r1, g1, dinv, b2, Wl, bl):
    return pl.pallas_call(
        _out_body,
        grid=(_GBO,),
        in_specs=[
            pl.BlockSpec((_NC, _BRO, _D), lambda i: (0, i, 0)),
            pl.BlockSpec((_BRO, _D), lambda i: (i, 0)),
            pl.BlockSpec((_BRO, 1), lambda i: (i, 0)),
            pl.BlockSpec((1, _D), lambda i: (0, 0)),
            pl.BlockSpec((_D, _D), lambda i: (0, 0)),
            pl.BlockSpec((1, _D), lambda i: (0, 0)),
        ],
        out_specs=pl.BlockSpec((_BRO, _D), lambda i: (i, 0)),
        out_shape=jax.ShapeDtypeStruct((_N, _D), jnp.float32),
    )(r1, g1, dinv, b2, Wl, bl)


def kernel(x, edge_index, W1, b1, W2, b2, Wl, bl):
    src = edge_index[0]
    dst = edge_index[1]
    npad_e = _EPAD - _E
    pad_idx = jnp.asarray(
        _N + (np.arange(npad_e) % (_NPAD - _N)).astype(np.int32))
    src2 = jnp.concatenate([src, pad_idx]).reshape(_EPAD // _K, _K)
    dst2 = jnp.concatenate([dst, pad_idx]).reshape(_EPAD // _K, _K)
    x_p = jnp.pad(x, ((0, _NPAD - _N), (0, 0)))
    zeros_d = jnp.asarray(np.zeros((_RPT, _D), np.float32))

    degp = _degree_count(dst2, zeros_d)
    dinv, g0 = _tc_prep(degp, x_p)
    r0 = _edge_aggregate(g0, src2, dst2, zeros_d)
    g1 = _tc_mid(r0, g0, dinv, W1, b1.reshape(1, _HID), W2)
    r1 = _edge_aggregate(g1, src2, dst2, zeros_d)
    return _tc_out(r1, g1, dinv, b2.reshape(1, _D), Wl, bl.reshape(1, _D))


# speedup vs baseline: 1.0358x; 1.0358x over previous
import functools

import jax
import jax.numpy as jnp
import numpy as np
from jax import lax
from jax.experimental import pallas as pl
from jax.experimental.pallas import tpu as pltpu
from jax.experimental.pallas import tpu_sc as plsc

_N = 10000
_D = 128
_HID = 256
_E = 320000

_NC = 2
_NS = 16
_NW = _NC * _NS
_NPAD = 10240
_RPT = _NPAD // _NS
_K = 128
_CPW = 80
_CPP = 16
_NP = _CPW // _CPP
_EPW = _K * _CPW
_EPAD = _NW * _EPW
_MROW = _E // _K
_MROWA = 2496
_PROW = (_EPAD - _MROWA * _K) // _K

_BR = 2048
_GB = _NPAD // _BR
_BRO = 2000
_GBO = _N // _BRO


def _sc_mesh():
    return plsc.VectorSubcoreMesh(core_axis_name="c", subcore_axis_name="s")


def _stage_idx(hbm, pad_hbm, base, buf):
    @pl.when(base + _CPP <= _MROWA)
    def _():
        pltpu.sync_copy(hbm.at[pl.ds(base, _CPP)], buf)

    @pl.when(base + _CPP > _MROWA)
    def _():
        pltpu.sync_copy(pad_hbm.at[pl.ds(base - _MROWA, _CPP)], buf)


@functools.partial(
    pl.kernel,
    mesh=_sc_mesh(),
    out_type=jax.ShapeDtypeStruct((_NC, _NPAD, _D), jnp.float32),
    scratch_types=[
        pltpu.VMEM((_CPP, _K), jnp.int32),
        pltpu.VMEM((_K, _D), jnp.float32),
        pltpu.VMEM_SHARED((_NPAD, _D), jnp.float32),
        pltpu.SemaphoreType.DMA((4,)),
    ],
)
def _degree_count(ei_hbm, pad_hbm, zeros_hbm, out_hbm, dst_v, ones_v, degb,
                  ssems):
    c = lax.axis_index("c")
    s = lax.axis_index("s")
    wid = s * _NC + c
    pltpu.sync_copy(zeros_hbm, degb.at[pl.ds(s * _RPT, _RPT)])
    ones = jnp.ones((16,), jnp.float32)

    def fill(i, carry):
        for w in range(_D // 16):
            ones_v[i, pl.ds(w * 16, 16)] = ones
        return carry

    lax.fori_loop(0, _K, fill, 0)
    plsc.subcore_barrier()

    for piece in range(_NP):
        base = wid * _CPW + piece * _CPP
        _stage_idx(ei_hbm.at[1], pad_hbm, base, dst_v)

        def body(i, carry):
            for k in range(4):
                j = 4 * i + k

                @pl.when(j >= 4)
                def _():
                    pltpu.make_async_copy(
                        ones_v, degb.at[dst_v.at[j - 4]], ssems.at[k]).wait()

                pltpu.async_copy(ones_v, degb.at[dst_v.at[j]], ssems.at[k],
                                 add=True)
            return carry

        lax.fori_loop(0, _CPP // 4, body, 0)
        for k in range(4):
            pltpu.make_async_copy(
                ones_v, degb.at[dst_v.at[_CPP - 4 + k]], ssems.at[k]).wait()
    plsc.subcore_barrier()
    pltpu.sync_copy(degb.at[pl.ds(s * _RPT, _RPT)],
                    out_hbm.at[c].at[pl.ds(s * _RPT, _RPT)])


@functools.partial(
    pl.kernel,
    mesh=_sc_mesh(),
    out_type=jax.ShapeDtypeStruct((_NC, _NPAD, _D), jnp.float32),
    scratch_types=[
        pltpu.VMEM((_CPP, _K), jnp.int32),
        pltpu.VMEM((_CPP, _K), jnp.int32),
        pltpu.VMEM((_K, _D), jnp.float32),
        pltpu.VMEM((_K, _D), jnp.float32),
        pltpu.VMEM_SHARED((_NPAD, _D), jnp.float32),
        pltpu.SemaphoreType.DMA,
        pltpu.SemaphoreType.DMA,
    ],
)
def _edge_aggregate(g_hbm, ei_hbm, pads_hbm, padd_hbm, zeros_hbm, out_hbm,
                    src_v, dst_v, rows0, rows1, agg, sem0, sem1):
    c = lax.axis_index("c")
    s = lax.axis_index("s")
    wid = s * _NC + c
    pltpu.sync_copy(zeros_hbm, agg.at[pl.ds(s * _RPT, _RPT)])
    plsc.subcore_barrier()

    def gather_start(j, buf, sem):
        pltpu.async_copy(g_hbm.at[src_v.at[j]], buf, sem)

    def gather_wait(j, buf, sem):
        pltpu.make_async_copy(g_hbm.at[src_v.at[j]], buf, sem).wait()

    def scatter_add(j, buf):
        pltpu.sync_copy(buf, agg.at[dst_v.at[j]], add=True)

    for piece in range(_NP):
        base = wid * _CPW + piece * _CPP
        _stage_idx(ei_hbm.at[0], pads_hbm, base, src_v)
        _stage_idx(ei_hbm.at[1], padd_hbm, base, dst_v)
        gather_start(0, rows0, sem0)

        def body(i, carry):
            j0 = 2 * i
            j1 = j0 + 1
            gather_start(j1, rows1, sem1)
            gather_wait(j0, rows0, sem0)
            scatter_add(j0, rows0)

            @pl.when(j1 + 1 < _CPP)
            def _():
                gather_start(j1 + 1, rows0, sem0)

            gather_wait(j1, rows1, sem1)
            scatter_add(j1, rows1)
            return carry

        lax.fori_loop(0, _CPP // 2, body, 0)
    plsc.subcore_barrier()
    pltpu.sync_copy(agg.at[pl.ds(s * _RPT, _RPT)],
                    out_hbm.at[c].at[pl.ds(s * _RPT, _RPT)])


def _prep_body(degp_ref, x_ref, dinv_ref, g0_ref):
    d = degp_ref[...]
    deg = d[0, :, :1] + d[1, :, :1] + 1.0
    dinv = lax.rsqrt(deg)
    dinv_ref[...] = dinv
    g0_ref[...] = x_ref[...] * dinv


def _tc_prep(degp, x):
    return pl.pallas_call(
        _prep_body,
        grid=(_GB,),
        in_specs=[
            pl.BlockSpec((_NC, _BR, _D), lambda i: (0, i, 0)),
            pl.BlockSpec((_BR, _D), lambda i: (i, 0)),
        ],
        out_specs=[
            pl.BlockSpec((_BR, 1), lambda i: (i, 0)),
            pl.BlockSpec((_BR, _D), lambda i: (i, 0)),
        ],
        out_shape=[
            jax.ShapeDtypeStruct((_NPAD, 1), jnp.float32),
            jax.ShapeDtypeStruct((_NPAD, _D), jnp.float32),
        ],
    )(degp, x)


def _mid_body(r_ref, g0_ref, dinv_ref, w1_ref, b1_ref, w2_ref, g1_ref):
    r = r_ref[...]
    di = dinv_ref[...]
    z = (r[0] + r[1] + g0_ref[...]) * di
    h = jnp.dot(z, w1_ref[...], preferred_element_type=jnp.float32)
    h = jnp.maximum(h + b1_ref[...], 0.0)
    t = jnp.dot(h, w2_ref[...], preferred_element_type=jnp.float32)
    g1_ref[...] = t * di


def _tc_mid(r0, g0, dinv, W1, b1, W2):
    return pl.pallas_call(
        _mid_body,
        grid=(_GB,),
        in_specs=[
            pl.BlockSpec((_NC, _BR, _D), lambda i: (0, i, 0)),
            pl.BlockSpec((_BR, _D), lambda i: (i, 0)),
            pl.BlockSpec((_BR, 1), lambda i: (i, 0)),
            pl.BlockSpec((_D, _HID), lambda i: (0, 0)),
            pl.BlockSpec((1, _HID), lambda i: (0, 0)),
            pl.BlockSpec((_HID, _D), lambda i: (0, 0)),
        ],
        out_specs=pl.BlockSpec((_BR, _D), lambda i: (i, 0)),
        out_shape=jax.ShapeDtypeStruct((_NPAD, _D), jnp.float32),
    )(r0, g0, dinv, W1, b1, W2)


def _out_body(r_ref, g1_ref, dinv_ref, b2_ref, wl_ref, bl_ref, out_ref):
    r = r_ref[...]
    di = dinv_ref[...]
    z = (r[0] + r[1] + g1_ref[...]) * di + b2_ref[...]
    h = jnp.maximum(z, 0.0)
    o = jnp.dot(h, wl_ref[...], preferred_element_type=jnp.float32)
    out_ref[...] = o + bl_ref[...]


def _tc_out(r1, g1, dinv, b2, Wl, bl):
    return pl.pallas_call(
        _out_body,
        grid=(_GBO,),
        in_specs=[
            pl.BlockSpec((_NC, _BRO, _D), lambda i: (0, i, 0)),
            pl.BlockSpec((_BRO, _D), lambda i: (i, 0)),
            pl.BlockSpec((_BRO, 1), lambda i: (i, 0)),
            pl.BlockSpec((1, _D), lambda i: (0, 0)),
            pl.BlockSpec((_D, _D), lambda i: (0, 0)),
            pl.BlockSpec((1, _D), lambda i: (0, 0)),
        ],
        out_specs=pl.BlockSpec((_BRO, _D), lambda i: (i, 0)),
        out_shape=jax.ShapeDtypeStruct((_N, _D), jnp.float32),
    )(r1, g1, dinv, b2, Wl, bl)


def kernel(x, edge_index, W1, b1, W2, b2, Wl, bl):
    ei3 = edge_index.reshape(2, _MROW, _K)
    ndum = _PROW - (_MROW - _MROWA)
    dummy = jnp.asarray(
        (_N + (np.arange(ndum * _K) % (_NPAD - _N))).astype(np.int32)
        .reshape(ndum, _K))
    pad_s = jnp.concatenate([ei3[0, _MROWA:], dummy])
    pad_d = jnp.concatenate([ei3[1, _MROWA:], dummy])
    zeros_d = jnp.asarray(np.zeros((_RPT, _D), np.float32))

    degp = _degree_count(ei3, pad_d, zeros_d)
    dinv, g0 = _tc_prep(degp, x)
    r0 = _edge_aggregate(g0, ei3, pad_s, pad_d, zeros_d)
    g1 = _tc_mid(r0, g0, dinv, W1, b1.reshape(1, _HID), W2)
    r1 = _edge_aggregate(g1, ei3, pad_s, pad_d, zeros_d)
    return _tc_out(r1, g1, dinv, b2.reshape(1, _D), Wl, bl.reshape(1, _D))


# speedup vs baseline: 1.0748x; 1.0377x over previous
import functools

import jax
import jax.numpy as jnp
import numpy as np
from jax import lax
from jax.experimental import pallas as pl
from jax.experimental.pallas import tpu as pltpu
from jax.experimental.pallas import tpu_sc as plsc

_N = 10000
_D = 128
_HID = 256
_E = 320000

_NC = 2
_NS = 16
_NW = _NC * _NS
_NPAD = 10240
_RPT = _NPAD // _NS
_K = 128
_CPW = 80
_CPP = 16
_NP = _CPW // _CPP
_EPW = _K * _CPW
_EPAD = _NW * _EPW
_MROW = _E // _K
_MROWA = 2496
_PROW = (_EPAD - _MROWA * _K) // _K

_BR = 2048
_GB = _NPAD // _BR
_BRO = 2000
_GBO = _N // _BRO


def _sc_mesh():
    return plsc.VectorSubcoreMesh(core_axis_name="c", subcore_axis_name="s")


def _stage_idx(hbm, pad_hbm, base, buf):
    @pl.when(base + _CPP <= _MROWA)
    def _():
        pltpu.sync_copy(hbm.at[pl.ds(base, _CPP)], buf)

    @pl.when(base + _CPP > _MROWA)
    def _():
        pltpu.sync_copy(pad_hbm.at[pl.ds(base - _MROWA, _CPP)], buf)


@functools.partial(
    pl.kernel,
    mesh=_sc_mesh(),
    out_type=jax.ShapeDtypeStruct((_NC, _NPAD, _D), jnp.float32),
    scratch_types=[
        pltpu.VMEM((_CPP, _K), jnp.int32),
        pltpu.VMEM((_K, _D), jnp.float32),
        pltpu.VMEM_SHARED((_NPAD, _D), jnp.float32),
        pltpu.SemaphoreType.DMA((4,)),
    ],
)
def _degree_count(ei_hbm, pad_hbm, zeros_hbm, out_hbm, dst_v, ones_v, degb,
                  ssems):
    c = lax.axis_index("c")
    s = lax.axis_index("s")
    wid = s * _NC + c
    pltpu.sync_copy(zeros_hbm, degb.at[pl.ds(s * _RPT, _RPT)])
    ones = jnp.ones((16,), jnp.float32)

    def fill(i, carry):
        for w in range(_D // 16):
            ones_v[i, pl.ds(w * 16, 16)] = ones
        return carry

    lax.fori_loop(0, _K, fill, 0)
    plsc.subcore_barrier()

    for piece in range(_NP):
        base = wid * _CPW + piece * _CPP
        _stage_idx(ei_hbm.at[1], pad_hbm, base, dst_v)

        def body(i, carry):
            for k in range(4):
                j = 4 * i + k

                @pl.when(j >= 4)
                def _():
                    pltpu.make_async_copy(
                        ones_v, degb.at[dst_v.at[j - 4]], ssems.at[k]).wait()

                pltpu.async_copy(ones_v, degb.at[dst_v.at[j]], ssems.at[k],
                                 add=True)
            return carry

        lax.fori_loop(0, _CPP // 4, body, 0)
        for k in range(4):
            pltpu.make_async_copy(
                ones_v, degb.at[dst_v.at[_CPP - 4 + k]], ssems.at[k]).wait()
    plsc.subcore_barrier()
    pltpu.sync_copy(degb.at[pl.ds(s * _RPT, _RPT)],
                    out_hbm.at[c].at[pl.ds(s * _RPT, _RPT)])


@functools.partial(
    pl.kernel,
    mesh=_sc_mesh(),
    out_type=jax.ShapeDtypeStruct((_NC, _NPAD, _D), jnp.float32),
    scratch_types=[
        pltpu.VMEM((2, _CPP, _K), jnp.int32),
        pltpu.VMEM((2, _CPP, _K), jnp.int32),
        pltpu.VMEM((_K, _D), jnp.float32),
        pltpu.VMEM((_K, _D), jnp.float32),
        pltpu.VMEM_SHARED((_NPAD, _D), jnp.float32),
        pltpu.SemaphoreType.DMA,
        pltpu.SemaphoreType.DMA,
        pltpu.SemaphoreType.DMA((4,)),
    ],
)
def _edge_aggregate(g_hbm, ei_hbm, pads_hbm, padd_hbm, zeros_hbm, out_hbm,
                    src_v, dst_v, rows0, rows1, agg, sem0, sem1, isems):
    c = lax.axis_index("c")
    s = lax.axis_index("s")
    wid = s * _NC + c
    wbase = wid * _CPW

    def stage_start(hbm, pad_hbm, base, buf, sem):
        @pl.when(base + _CPP <= _MROWA)
        def _():
            pltpu.async_copy(hbm.at[pl.ds(base, _CPP)], buf, sem)

        @pl.when(base + _CPP > _MROWA)
        def _():
            pltpu.async_copy(pad_hbm.at[pl.ds(base - _MROWA, _CPP)], buf, sem)

    def stage_wait(hbm, pad_hbm, base, buf, sem):
        @pl.when(base + _CPP <= _MROWA)
        def _():
            pltpu.make_async_copy(hbm.at[pl.ds(base, _CPP)], buf, sem).wait()

        @pl.when(base + _CPP > _MROWA)
        def _():
            pltpu.make_async_copy(
                pad_hbm.at[pl.ds(base - _MROWA, _CPP)], buf, sem).wait()

    stage_start(ei_hbm.at[0], pads_hbm, wbase, src_v.at[0], isems.at[0])
    stage_start(ei_hbm.at[1], padd_hbm, wbase, dst_v.at[0], isems.at[2])
    pltpu.sync_copy(zeros_hbm, agg.at[pl.ds(s * _RPT, _RPT)])
    plsc.subcore_barrier()

    def gather_start(sv, j, buf, sem):
        pltpu.async_copy(g_hbm.at[sv.at[j]], buf, sem)

    def gather_wait(sv, j, buf, sem):
        pltpu.make_async_copy(g_hbm.at[sv.at[j]], buf, sem).wait()

    def scatter_add(dv, j, buf):
        pltpu.sync_copy(buf, agg.at[dv.at[j]], add=True)

    for piece in range(_NP):
        p = piece % 2
        pn = 1 - p
        base = wbase + piece * _CPP
        sv = src_v.at[p]
        dv = dst_v.at[p]
        stage_wait(ei_hbm.at[0], pads_hbm, base, sv, isems.at[p])
        stage_wait(ei_hbm.at[1], padd_hbm, base, dv, isems.at[2 + p])
        if piece + 1 < _NP:
            nbase = base + _CPP
            stage_start(ei_hbm.at[0], pads_hbm, nbase, src_v.at[pn],
                        isems.at[pn])
            stage_start(ei_hbm.at[1], padd_hbm, nbase, dst_v.at[pn],
                        isems.at[2 + pn])
        gather_start(sv, 0, rows0, sem0)

        def body(i, carry):
            j0 = 2 * i
            j1 = j0 + 1
            gather_start(sv, j1, rows1, sem1)
            gather_wait(sv, j0, rows0, sem0)
            scatter_add(dv, j0, rows0)

            @pl.when(j1 + 1 < _CPP)
            def _():
                gather_start(sv, j1 + 1, rows0, sem0)

            gather_wait(sv, j1, rows1, sem1)
            scatter_add(dv, j1, rows1)
            return carry

        lax.fori_loop(0, _CPP // 2, body, 0)
    plsc.subcore_barrier()
    pltpu.sync_copy(agg.at[pl.ds(s * _RPT, _RPT)],
                    out_hbm.at[c].at[pl.ds(s * _RPT, _RPT)])


def _prep_body(degp_ref, x_ref, dinv_ref, g0_ref):
    d = degp_ref[...]
    deg = d[0, :, :1] + d[1, :, :1] + 1.0
    dinv = lax.rsqrt(deg)
    dinv_ref[...] = dinv
    g0_ref[...] = x_ref[...] * dinv


def _tc_prep(degp, x):
    return pl.pallas_call(
        _prep_body,
        grid=(_GB,),
        in_specs=[
            pl.BlockSpec((_NC, _BR, _D), lambda i: (0, i, 0)),
            pl.BlockSpec((_BR, _D), lambda i: (i, 0)),
        ],
        out_specs=[
            pl.BlockSpec((_BR, 1), lambda i: (i, 0)),
            pl.BlockSpec((_BR, _D), lambda i: (i, 0)),
        ],
        out_shape=[
            jax.ShapeDtypeStruct((_NPAD, 1), jnp.float32),
            jax.ShapeDtypeStruct((_NPAD, _D), jnp.float32),
        ],
    )(degp, x)


def _mid_body(r_ref, g0_ref, dinv_ref, w1_ref, b1_ref, w2_ref, g1_ref):
    r = r_ref[...]
    di = dinv_ref[...]
    z = (r[0] + r[1] + g0_ref[...]) * di
    h = jnp.dot(z, w1_ref[...], preferred_element_type=jnp.float32)
    h = jnp.maximum(h + b1_ref[...], 0.0)
    t = jnp.dot(h, w2_ref[...], preferred_element_type=jnp.float32)
    g1_ref[...] = t * di


def _tc_mid(r0, g0, dinv, W1, b1, W2):
    return pl.pallas_call(
        _mid_body,
        grid=(_GB,),
        in_specs=[
            pl.BlockSpec((_NC, _BR, _D), lambda i: (0, i, 0)),
            pl.BlockSpec((_BR, _D), lambda i: (i, 0)),
            pl.BlockSpec((_BR, 1), lambda i: (i, 0)),
            pl.BlockSpec((_D, _HID), lambda i: (0, 0)),
            pl.BlockSpec((1, _HID), lambda i: (0, 0)),
            pl.BlockSpec((_HID, _D), lambda i: (0, 0)),
        ],
        out_specs=pl.BlockSpec((_BR, _D), lambda i: (i, 0)),
        out_shape=jax.ShapeDtypeStruct((_NPAD, _D), jnp.float32),
    )(r0, g0, dinv, W1, b1, W2)


def _out_body(r_ref, g1_ref, dinv_ref, b2_ref, wl_ref, bl_ref, out_ref):
    r = r_ref[...]
    di = dinv_ref[...]
    z = (r[0] + r[1] + g1_ref[...]) * di + b2_ref[...]
    h = jnp.maximum(z, 0.0)
    o = jnp.dot(h, wl_ref[...], preferred_element_type=jnp.float32)
    out_ref[...] = o + bl_ref[...]


def _tc_out(r1, g1, dinv, b2, Wl, bl):
    return pl.pallas_call(
        _out_body,
        grid=(_GBO,),
        in_specs=[
            pl.BlockSpec((_NC, _BRO, _D), lambda i: (0, i, 0)),
            pl.BlockSpec((_BRO, _D), lambda i: (i, 0)),
            pl.BlockSpec((_BRO, 1), lambda i: (i, 0)),
            pl.BlockSpec((1, _D), lambda i: (0, 0)),
            pl.BlockSpec((_D, _D), lambda i: (0, 0)),
            pl.BlockSpec((1, _D), lambda i: (0, 0)),
        ],
        out_specs=pl.BlockSpec((_BRO, _D), lambda i: (i, 0)),
        out_shape=jax.ShapeDtypeStruct((_N, _D), jnp.float32),
    )(r1, g1, dinv, b2, Wl, bl)


def kernel(x, edge_index, W1, b1, W2, b2, Wl, bl):
    ei3 = edge_index.reshape(2, _MROW, _K)
    ndum = _PROW - (_MROW - _MROWA)
    dummy = jnp.asarray(
        (_N + (np.arange(ndum * _K) % (_NPAD - _N))).astype(np.int32)
        .reshape(ndum, _K))
    pad_s = jnp.concatenate([ei3[0, _MROWA:], dummy])
    pad_d = jnp.concatenate([ei3[1, _MROWA:], dummy])
    zeros_d = jnp.asarray(np.zeros((_RPT, _D), np.float32))

    degp = _degree_count(ei3, pad_d, zeros_d)
    dinv, g0 = _tc_prep(degp, x)
    r0 = _edge_aggregate(g0, ei3, pad_s, pad_d, zeros_d)
    g1 = _tc_mid(r0, g0, dinv, W1, b1.reshape(1, _HID), W2)
    r1 = _edge_aggregate(g1, ei3, pad_s, pad_d, zeros_d)
    return _tc_out(r1, g1, dinv, b2.reshape(1, _D), Wl, bl.reshape(1, _D))
